# baseline (speedup 1.0000x reference)
import functools

import jax
import jax.numpy as jnp
from jax import lax
from jax.experimental import pallas as pl
from jax.experimental.pallas import tpu as pltpu
from jax.experimental.pallas import tpu_sc as plsc

_N = 8192
_K = 16
_R = 256
_RM = 256


_C = 1024
_BIG = 1 << 20


def _topk_body(c0b_ref, nact_ref, xb_ref, xfc_ref, brow_ref, bcolc_ref,
               idx_ref, cval_ref, cgid_ref, *, d):
    i = pl.program_id(0)
    j = pl.program_id(1)
    active = (j == 0) | (j - 1 < nact_ref[i])

    @pl.when(active)
    def _():
        chunk = jnp.where(j == 0, 0,
                          jnp.minimum(c0b_ref[i] + j - 1,
                                      c0b_ref[i] + nact_ref[i] - 1))
        xb = xb_ref[...]
        xfc = xfc_ref[...]
        sqc = lax.dot_general(
            jnp.ones((8, d), jnp.float32), xfc * xfc,
            (((1,), (1,)), ((), ())),
            precision=lax.Precision.HIGHEST,
            preferred_element_type=jnp.float32)[0:1, :]
        score = lax.dot_general(xb.astype(jnp.bfloat16),
                                xfc.astype(jnp.bfloat16),
                                (((1,), (1,)), ((), ())),
                                preferred_element_type=jnp.float32)
        score = sqc - 2.0 * score
        same = brow_ref[...] == bcolc_ref[...]
        score = jnp.where(same, score, jnp.inf)
        fbig = jnp.float32(_BIG)
        gid = ((chunk * _C).astype(jnp.float32) +
               lax.broadcasted_iota(jnp.int32, score.shape, 1
                                    ).astype(jnp.float32))

        cv = jnp.where(j == 0, jnp.inf, cval_ref[...])
        cg = jnp.where(j == 0, fbig, cgid_ref[...])
        val = jnp.concatenate([cv, score], axis=1)
        gids = jnp.concatenate([cg, gid], axis=1)
        new_v, new_g = [], []
        for _ in range(_K):
            m = jnp.min(val, axis=1, keepdims=True)
            g = jnp.min(jnp.where(val == m, gids, fbig), axis=1,
                        keepdims=True)
            new_v.append(m)
            new_g.append(g)
            hit = gids == g
            val = jnp.where(hit, jnp.inf, val)
            gids = jnp.where(hit, fbig, gids)
        cval_ref[...] = jnp.concatenate(new_v, axis=1)
        ng = jnp.concatenate(new_g, axis=1)
        cgid_ref[...] = ng
        idx_ref[...] = ng.astype(jnp.int32)


@functools.lru_cache(maxsize=None)
def _topk_call(d):
    nb = _N // _R

    def chunk_map(i, j, c0b, nact):
        return (jnp.where(j == 0, 0,
                          jnp.minimum(c0b[i] + j - 1, c0b[i] + nact[i] - 1)),
                0)

    return pl.pallas_call(
        functools.partial(_topk_body, d=d),
        grid_spec=pltpu.PrefetchScalarGridSpec(
            num_scalar_prefetch=2,
            grid=(nb, 1 + _N // _C),
            in_specs=[
                pl.BlockSpec((_R, d), lambda i, j, c0b, nact: (i, 0)),
                pl.BlockSpec((_C, d), chunk_map),
                pl.BlockSpec((_R, 1), lambda i, j, c0b, nact: (i, 0)),
                pl.BlockSpec((1, _C),
                             lambda i, j, c0b, nact: (0, chunk_map(i, j, c0b, nact)[0])),
            ],
            out_specs=pl.BlockSpec((_R, _K), lambda i, j, c0b, nact: (i, 0)),
            scratch_shapes=[pltpu.VMEM((_R, _K), jnp.float32),
                            pltpu.VMEM((_R, _K), jnp.float32)],
        ),
        out_shape=jax.ShapeDtypeStruct((_N, _K), jnp.int32),
        compiler_params=pltpu.CompilerParams(
            dimension_semantics=("arbitrary", "arbitrary")),
    )


_CH = 128
_NHALF = _N // 2


@functools.lru_cache(maxsize=None)
def _sc_gather(d):
    mesh = plsc.VectorSubcoreMesh(core_axis_name="c", subcore_axis_name="s")

    @functools.partial(
        pl.kernel, mesh=mesh,
        out_type=jax.ShapeDtypeStruct((_K, _N, d), jnp.float32),
        scratch_types=[
            pltpu.VMEM((_CH,), jnp.int32),
            pltpu.VMEM((_CH,), jnp.int32),
            pltpu.VMEM((_CH, d), jnp.float32),
            pltpu.VMEM((_CH, d), jnp.float32),
            pltpu.SemaphoreType.DMA,
            pltpu.SemaphoreType.DMA,
        ])
    def k(x_hbm, idxT_hbm, out_hbm, i0, i1, st0, st1, s0, s1):
        wid = lax.axis_index("s") * 2 + lax.axis_index("c")
        kk = wid // 2
        base = (wid % 2) * _NHALF
        nch = _NHALF // _CH

        def load(c, iv, stv, sv):
            pltpu.sync_copy(idxT_hbm.at[kk, pl.ds(base + c * _CH, _CH)], iv)
            pltpu.async_copy(x_hbm.at[iv], stv, sv)

        load(0, i0, st0, s0)
        load(1, i1, st1, s1)

        def body(g, carry):
            for bsel, (iv, stv, sv) in enumerate(((i0, st0, s0),
                                                  (i1, st1, s1))):
                c = 2 * g + bsel
                pltpu.make_async_copy(x_hbm.at[pl.ds(0, _CH)], stv, sv).wait()
                pltpu.sync_copy(stv, out_hbm.at[kk, pl.ds(base + c * _CH, _CH)])
                nc = c + 2

                @pl.when(nc < nch)
                def _():
                    load(nc, iv, stv, sv)
            return carry

        lax.fori_loop(0, nch // 2, body, 0)

    return k


def _msg_body(xb_ref, xj_ref, Wat_ref, Wbt_ref, b_ref, o_ref, *, relu):
    xb = xb_ref[...]
    apart = lax.dot_general(xb.astype(jnp.bfloat16), Wat_ref[...],
                            (((1,), (0,)), ((), ())),
                            preferred_element_type=jnp.float32)
    m = None
    for k in range(_K):
        dm = (xj_ref[k] - xb).astype(jnp.bfloat16)
        b2 = lax.dot_general(dm, Wbt_ref[...], (((1,), (0,)), ((), ())),
                             preferred_element_type=jnp.float32)
        m = b2 if m is None else jnp.maximum(m, b2)
    res = apart + b_ref[...] + m
    if relu:
        res = jnp.maximum(res, 0.0)
    o_ref[...] = res


@functools.lru_cache(maxsize=None)
def _msg_call(d, dout, relu):
    return pl.pallas_call(
        functools.partial(_msg_body, relu=relu),
        grid=(_N // _RM,),
        in_specs=[
            pl.BlockSpec((_RM, d), lambda i: (i, 0)),
            pl.BlockSpec((_K, _RM, d), lambda i: (0, i, 0)),
            pl.BlockSpec((d, dout), lambda i: (0, 0)),
            pl.BlockSpec((d, dout), lambda i: (0, 0)),
            pl.BlockSpec((1, dout), lambda i: (0, 0)),
        ],
        out_specs=pl.BlockSpec((_RM, dout), lambda i: (i, 0)),
        out_shape=jax.ShapeDtypeStruct((_N, dout), jnp.float32),
        compiler_params=pltpu.CompilerParams(
            dimension_semantics=("arbitrary",)),
    )


def _layer(x, brow, bcol, c0b, nact, W, b, relu):
    d = x.shape[1]
    dout = W.shape[0]
    idx = _topk_call(d)(c0b, nact, x, x, brow, bcol)
    idxT = jnp.transpose(idx)
    xj = _sc_gather(d)(x, idxT)
    Wat16 = jnp.transpose(W[:, :d]).astype(jnp.bfloat16)
    Wbt16 = jnp.transpose(W[:, d:]).astype(jnp.bfloat16)
    return _msg_call(d, dout, relu)(x, xj, Wat16, Wbt16, b.reshape(1, dout))


def kernel(x, batch, W1, b1, W2, b2, W3, b3):
    b32 = batch.astype(jnp.int32)
    brow = b32.reshape(_N, 1)
    bcol = b32.reshape(1, _N)
    segs = jnp.arange(8, dtype=jnp.int32)
    starts = jnp.searchsorted(b32, segs, side="left").astype(jnp.int32)
    ends = jnp.searchsorted(b32, segs, side="right").astype(jnp.int32)
    s0 = b32[::_R]
    s1 = b32[_R - 1::_R]
    c0 = starts[s0]
    c1 = ends[s1]
    c0b = (c0 // _C).astype(jnp.int32)
    nact = ((c1 - 1) // _C - c0b + 1).astype(jnp.int32)
    h = _layer(x, brow, bcol, c0b, nact, W1, b1, True)
    h = _layer(h, brow, bcol, c0b, nact, W2, b2, True)
    h = _layer(h, brow, bcol, c0b, nact, W3, b3, False)
    return h


# speedup vs baseline: 1.1123x; 1.1123x over previous
import functools

import jax
import jax.numpy as jnp
from jax import lax
from jax.experimental import pallas as pl
from jax.experimental.pallas import tpu as pltpu
from jax.experimental.pallas import tpu_sc as plsc

_N = 8192
_K = 16
_R = 256
_RM = 256


_C = 512
_BIG = 1 << 20


def _topk_body(c0b_ref, nact_ref, xb_ref, xfc_ref, brow_ref, bcolc_ref,
               idx_ref, cval_ref, cgid_ref, *, d):
    i = pl.program_id(0)
    j = pl.program_id(1)
    active = (j == 0) | (j - 1 < nact_ref[i])

    @pl.when(active)
    def _():
        chunk = jnp.where(j == 0, 0,
                          jnp.minimum(c0b_ref[i] + j - 1,
                                      c0b_ref[i] + nact_ref[i] - 1))
        xb = xb_ref[...]
        xfc = xfc_ref[...]
        sqc = lax.dot_general(
            jnp.ones((8, d), jnp.float32), xfc * xfc,
            (((1,), (1,)), ((), ())),
            precision=lax.Precision.HIGHEST,
            preferred_element_type=jnp.float32)[0:1, :]
        score = lax.dot_general(xb.astype(jnp.bfloat16),
                                xfc.astype(jnp.bfloat16),
                                (((1,), (1,)), ((), ())),
                                preferred_element_type=jnp.float32)
        score = sqc - 2.0 * score
        same = brow_ref[...] == bcolc_ref[...]
        score = jnp.where(same, score, jnp.inf)
        fbig = jnp.float32(_BIG)
        gid = ((chunk * _C).astype(jnp.float32) +
               lax.broadcasted_iota(jnp.int32, score.shape, 1
                                    ).astype(jnp.float32))

        cv = jnp.where(j == 0, jnp.inf, cval_ref[...])
        cg = jnp.where(j == 0, fbig, cgid_ref[...])
        val = jnp.concatenate([cv, score], axis=1)
        gids = jnp.concatenate([cg, gid], axis=1)
        new_v, new_g = [], []
        for _ in range(_K):
            m = jnp.min(val, axis=1, keepdims=True)
            g = jnp.min(jnp.where(val == m, gids, fbig), axis=1,
                        keepdims=True)
            new_v.append(m)
            new_g.append(g)
            hit = gids == g
            val = jnp.where(hit, jnp.inf, val)
            gids = jnp.where(hit, fbig, gids)
        cval_ref[...] = jnp.concatenate(new_v, axis=1)
        ng = jnp.concatenate(new_g, axis=1)
        cgid_ref[...] = ng
        idx_ref[...] = ng.astype(jnp.int32)


@functools.lru_cache(maxsize=None)
def _topk_call(d):
    nb = _N // _R

    def chunk_map(i, j, c0b, nact):
        return (jnp.where(j == 0, 0,
                          jnp.minimum(c0b[i] + j - 1, c0b[i] + nact[i] - 1)),
                0)

    return pl.pallas_call(
        functools.partial(_topk_body, d=d),
        grid_spec=pltpu.PrefetchScalarGridSpec(
            num_scalar_prefetch=2,
            grid=(nb, 1 + _N // _C),
            in_specs=[
                pl.BlockSpec((_R, d), lambda i, j, c0b, nact: (i, 0)),
                pl.BlockSpec((_C, d), chunk_map),
                pl.BlockSpec((_R, 1), lambda i, j, c0b, nact: (i, 0)),
                pl.BlockSpec((1, _C),
                             lambda i, j, c0b, nact: (0, chunk_map(i, j, c0b, nact)[0])),
            ],
            out_specs=pl.BlockSpec((_R, _K), lambda i, j, c0b, nact: (i, 0)),
            scratch_shapes=[pltpu.VMEM((_R, _K), jnp.float32),
                            pltpu.VMEM((_R, _K), jnp.float32)],
        ),
        out_shape=jax.ShapeDtypeStruct((_N, _K), jnp.int32),
        compiler_params=pltpu.CompilerParams(
            dimension_semantics=("arbitrary", "arbitrary")),
    )


_CH = 128
_NHALF = _N // 2


@functools.lru_cache(maxsize=None)
def _sc_gather(d):
    mesh = plsc.VectorSubcoreMesh(core_axis_name="c", subcore_axis_name="s")

    @functools.partial(
        pl.kernel, mesh=mesh,
        out_type=jax.ShapeDtypeStruct((_K, _N, d), jnp.float32),
        scratch_types=[
            pltpu.VMEM((_CH,), jnp.int32),
            pltpu.VMEM((_CH,), jnp.int32),
            pltpu.VMEM((_CH, d), jnp.float32),
            pltpu.VMEM((_CH, d), jnp.float32),
            pltpu.SemaphoreType.DMA,
            pltpu.SemaphoreType.DMA,
        ])
    def k(x_hbm, idxT_hbm, out_hbm, i0, i1, st0, st1, s0, s1):
        wid = lax.axis_index("s") * 2 + lax.axis_index("c")
        kk = wid // 2
        base = (wid % 2) * _NHALF
        nch = _NHALF // _CH

        def load(c, iv, stv, sv):
            pltpu.sync_copy(idxT_hbm.at[kk, pl.ds(base + c * _CH, _CH)], iv)
            pltpu.async_copy(x_hbm.at[iv], stv, sv)

        load(0, i0, st0, s0)
        load(1, i1, st1, s1)

        def body(g, carry):
            for bsel, (iv, stv, sv) in enumerate(((i0, st0, s0),
                                                  (i1, st1, s1))):
                c = 2 * g + bsel
                pltpu.make_async_copy(x_hbm.at[pl.ds(0, _CH)], stv, sv).wait()
                pltpu.sync_copy(stv, out_hbm.at[kk, pl.ds(base + c * _CH, _CH)])
                nc = c + 2

                @pl.when(nc < nch)
                def _():
                    load(nc, iv, stv, sv)
            return carry

        lax.fori_loop(0, nch // 2, body, 0)

    return k


def _msg_body(xb_ref, xj_ref, Wat_ref, Wbt_ref, b_ref, o_ref, *, relu):
    xb = xb_ref[...]
    apart = lax.dot_general(xb.astype(jnp.bfloat16), Wat_ref[...],
                            (((1,), (0,)), ((), ())),
                            preferred_element_type=jnp.float32)
    m = None
    for k in range(_K):
        dm = (xj_ref[k] - xb).astype(jnp.bfloat16)
        b2 = lax.dot_general(dm, Wbt_ref[...], (((1,), (0,)), ((), ())),
                             preferred_element_type=jnp.float32)
        m = b2 if m is None else jnp.maximum(m, b2)
    res = apart + b_ref[...] + m
    if relu:
        res = jnp.maximum(res, 0.0)
    o_ref[...] = res


@functools.lru_cache(maxsize=None)
def _msg_call(d, dout, relu):
    return pl.pallas_call(
        functools.partial(_msg_body, relu=relu),
        grid=(_N // _RM,),
        in_specs=[
            pl.BlockSpec((_RM, d), lambda i: (i, 0)),
            pl.BlockSpec((_K, _RM, d), lambda i: (0, i, 0)),
            pl.BlockSpec((d, dout), lambda i: (0, 0)),
            pl.BlockSpec((d, dout), lambda i: (0, 0)),
            pl.BlockSpec((1, dout), lambda i: (0, 0)),
        ],
        out_specs=pl.BlockSpec((_RM, dout), lambda i: (i, 0)),
        out_shape=jax.ShapeDtypeStruct((_N, dout), jnp.float32),
        compiler_params=pltpu.CompilerParams(
            dimension_semantics=("arbitrary",)),
    )


def _layer(x, brow, bcol, c0b, nact, W, b, relu):
    d = x.shape[1]
    dout = W.shape[0]
    idx = _topk_call(d)(c0b, nact, x, x, brow, bcol)
    idxT = jnp.transpose(idx)
    xj = _sc_gather(d)(x, idxT)
    Wat16 = jnp.transpose(W[:, :d]).astype(jnp.bfloat16)
    Wbt16 = jnp.transpose(W[:, d:]).astype(jnp.bfloat16)
    return _msg_call(d, dout, relu)(x, xj, Wat16, Wbt16, b.reshape(1, dout))


def kernel(x, batch, W1, b1, W2, b2, W3, b3):
    b32 = batch.astype(jnp.int32)
    brow = b32.reshape(_N, 1)
    bcol = b32.reshape(1, _N)
    segs = jnp.arange(8, dtype=jnp.int32)
    starts = jnp.searchsorted(b32, segs, side="left").astype(jnp.int32)
    ends = jnp.searchsorted(b32, segs, side="right").astype(jnp.int32)
    s0 = b32[::_R]
    s1 = b32[_R - 1::_R]
    c0 = starts[s0]
    c1 = ends[s1]
    c0b = (c0 // _C).astype(jnp.int32)
    nact = ((c1 - 1) // _C - c0b + 1).astype(jnp.int32)
    h = _layer(x, brow, bcol, c0b, nact, W1, b1, True)
    h = _layer(h, brow, bcol, c0b, nact, W2, b2, True)
    h = _layer(h, brow, bcol, c0b, nact, W3, b3, False)
    return h
